# baseline (speedup 1.0000x reference)
import jax
import jax.numpy as jnp
from jax import lax
from jax.experimental import pallas as pl
from jax.experimental.pallas import tpu as pltpu
from jax.experimental.pallas import tpu_sc as plsc

N_RES = 65
C_Z = 128
LANES = 16
N_PAD = 80
N_CHUNKS = N_PAD // LANES
ROWS_PER_WORKER = 5
N_WORKERS = 13


def _relpos_body(ri_hbm, tab_hbm, out_hbm, ri_v, sem_in, sem_g):
    wid = lax.axis_index("s")

    @pl.when(wid < N_WORKERS)
    def _work():
        pltpu.async_copy(ri_hbm, ri_v.at[pl.ds(0, N_RES)], sem_in).wait()

        lane = lax.iota(jnp.int32, LANES)
        lane_f = lane.astype(jnp.float32)
        inf16 = jnp.full((LANES,), jnp.inf, jnp.float32)
        rots = [(lane + s) % LANES for s in (1, 2, 4, 8)]
        base = wid * ROWS_PER_WORKER

        chunks = [ri_v[pl.ds(k * LANES, LANES)] for k in range(N_CHUNKS)]

        copies = []
        for r in range(ROWS_PER_WORKER):
            i = base + r
            ri_i = ri_v[pl.ds(i, LANES)][0]
            best_val = inf16
            best_j = jnp.zeros((LANES,), jnp.int32)
            for k in range(N_CHUNKS):
                jvec = lane + (k * LANES)
                t = jnp.abs(chunks[k] - ri_i - (lane_f + float(k * LANES - 32)))
                invalid = (jvec == i) | (jvec >= N_RES)
                t = jnp.where(invalid, inf16, t)
                upd = t < best_val
                best_val = jnp.where(upd, t, best_val)
                best_j = jnp.where(upd, jvec, best_j)
            for rot in rots:
                sh_v = jnp.take(best_val, rot)
                sh_j = jnp.take(best_j, rot)
                better = sh_v < best_val
                tie = (sh_v == best_val) & (sh_j < best_j)
                best_val = jnp.where(better, sh_v, best_val)
                best_j = jnp.where(better | tie, sh_j, best_j)
            idx = best_j[0]
            copies.append(
                pltpu.async_copy(
                    tab_hbm.at[pl.ds(idx * C_Z, C_Z)],
                    out_hbm.at[pl.ds(i * C_Z, C_Z)],
                    sem_g,
                )
            )
        for cp in copies:
            cp.wait()


def kernel(residue_index, W, b):
    table = (W.T + b[None, :]).reshape(N_RES * C_Z)
    mesh = plsc.VectorSubcoreMesh(
        core_axis_name="c", subcore_axis_name="s", num_cores=1
    )
    out = pl.kernel(
        _relpos_body,
        mesh=mesh,
        out_type=jax.ShapeDtypeStruct((N_RES * C_Z,), jnp.float32),
        scratch_types=[
            pltpu.VMEM((N_PAD,), jnp.float32),
            pltpu.SemaphoreType.DMA,
            pltpu.SemaphoreType.DMA,
        ],
    )(residue_index, table)
    return out.reshape(N_RES, C_Z)


# speedup vs baseline: 1.0021x; 1.0021x over previous
import jax
import jax.numpy as jnp
from jax import lax
from jax.experimental import pallas as pl
from jax.experimental.pallas import tpu as pltpu
from jax.experimental.pallas import tpu_sc as plsc

N_RES = 65
C_Z = 128
LANES = 16
N_PAD = 80
N_CHUNKS = N_PAD // LANES
ROWS_PER_WORKER = 5
N_WORKERS = 13
C_CHUNKS = C_Z // LANES


def _relpos_body(ri_hbm, wt_hbm, b_hbm, out_hbm,
                 ri_v, b_v, rows_v, out_v, sem_in, sem_g, sem_o):
    wid = lax.axis_index("s")

    @pl.when(wid < N_WORKERS)
    def _work():
        cp_ri = pltpu.async_copy(ri_hbm, ri_v.at[pl.ds(0, N_RES)], sem_in)
        cp_b = pltpu.async_copy(b_hbm, b_v, sem_in)
        cp_ri.wait()

        lane = lax.iota(jnp.int32, LANES)
        lane_f = lane.astype(jnp.float32)
        inf16 = jnp.full((LANES,), jnp.inf, jnp.float32)
        rots = [(lane + s) % LANES for s in (1, 2, 4, 8)]
        base = wid * ROWS_PER_WORKER

        chunks = [ri_v[pl.ds(k * LANES, LANES)] for k in range(N_CHUNKS)]

        gathers = []
        for r in range(ROWS_PER_WORKER):
            i = base + r
            ri_i = ri_v[pl.ds(i, LANES)][0]
            best_val = inf16
            best_j = jnp.zeros((LANES,), jnp.int32)
            for k in range(N_CHUNKS):
                jvec = lane + (k * LANES)
                t = jnp.abs(chunks[k] - ri_i - (lane_f + float(k * LANES - 32)))
                invalid = (jvec == i) | (jvec >= N_RES)
                t = jnp.where(invalid, inf16, t)
                upd = t < best_val
                best_val = jnp.where(upd, t, best_val)
                best_j = jnp.where(upd, jvec, best_j)
            for rot in rots:
                sh_v = jnp.take(best_val, rot)
                sh_j = jnp.take(best_j, rot)
                better = sh_v < best_val
                tie = (sh_v == best_val) & (sh_j < best_j)
                best_val = jnp.where(better, sh_v, best_val)
                best_j = jnp.where(better | tie, sh_j, best_j)
            idx = best_j[0]
            gathers.append(
                pltpu.async_copy(
                    wt_hbm.at[pl.ds(idx * C_Z, C_Z)],
                    rows_v.at[pl.ds(r * C_Z, C_Z)],
                    sem_g,
                )
            )
        cp_b.wait()
        stores = []
        for r in range(ROWS_PER_WORKER):
            gathers[r].wait()
            for c in range(C_CHUNKS):
                out_v[pl.ds(r * C_Z + c * LANES, LANES)] = (
                    rows_v[pl.ds(r * C_Z + c * LANES, LANES)]
                    + b_v[pl.ds(c * LANES, LANES)]
                )
            stores.append(
                pltpu.async_copy(
                    out_v.at[pl.ds(r * C_Z, C_Z)],
                    out_hbm.at[pl.ds((base + r) * C_Z, C_Z)],
                    sem_o,
                )
            )
        for cp in stores:
            cp.wait()


def kernel(residue_index, W, b):
    wt_flat = W.T.reshape(N_RES * C_Z)
    mesh = plsc.VectorSubcoreMesh(
        core_axis_name="c", subcore_axis_name="s", num_cores=1
    )
    out = pl.kernel(
        _relpos_body,
        mesh=mesh,
        out_type=jax.ShapeDtypeStruct((N_RES * C_Z,), jnp.float32),
        scratch_types=[
            pltpu.VMEM((N_PAD,), jnp.float32),
            pltpu.VMEM((C_Z,), jnp.float32),
            pltpu.VMEM((ROWS_PER_WORKER * C_Z,), jnp.float32),
            pltpu.VMEM((ROWS_PER_WORKER * C_Z,), jnp.float32),
            pltpu.SemaphoreType.DMA,
            pltpu.SemaphoreType.DMA,
            pltpu.SemaphoreType.DMA,
        ],
    )(residue_index, wt_flat, b)
    return out.reshape(N_RES, C_Z)


# speedup vs baseline: 1.0820x; 1.0797x over previous
import jax
import jax.numpy as jnp
from jax import lax
from jax.experimental import pallas as pl
from jax.experimental.pallas import tpu as pltpu
from jax.experimental.pallas import tpu_sc as plsc

N_RES = 65
C_Z = 128
LANES = 16
N_PAD = 80
N_CHUNKS = N_PAD // LANES
ROWS_PER_WORKER = 5
N_WORKERS = 13
C_CHUNKS = C_Z // LANES


def _relpos_body(ri_hbm, wt_hbm, b_hbm, out_hbm,
                 ri_v, b_v, wt_v, out_v, sem_ri, sem_t, sem_o):
    wid = lax.axis_index("s")

    @pl.when(wid < N_WORKERS)
    def _work():
        cp_ri = pltpu.async_copy(ri_hbm, ri_v.at[pl.ds(0, N_RES)], sem_ri)
        cp_wt = pltpu.async_copy(wt_hbm, wt_v, sem_t)
        cp_b = pltpu.async_copy(b_hbm, b_v, sem_t)
        cp_ri.wait()

        lane = lax.iota(jnp.int32, LANES)
        lane_f = lane.astype(jnp.float32)
        inf16 = jnp.full((LANES,), jnp.inf, jnp.float32)
        rots = [(lane + s) % LANES for s in (1, 2, 4, 8)]
        base = wid * ROWS_PER_WORKER

        chunks = [ri_v[pl.ds(k * LANES, LANES)] for k in range(N_CHUNKS)]

        idxs = []
        for r in range(ROWS_PER_WORKER):
            i = base + r
            ri_i = ri_v[pl.ds(i, LANES)][0]
            best_val = inf16
            best_j = jnp.zeros((LANES,), jnp.int32)
            for k in range(N_CHUNKS):
                jvec = lane + (k * LANES)
                t = jnp.abs(chunks[k] - ri_i - (lane_f + float(k * LANES - 32)))
                invalid = (jvec == i) | (jvec >= N_RES)
                t = jnp.where(invalid, inf16, t)
                upd = t < best_val
                best_val = jnp.where(upd, t, best_val)
                best_j = jnp.where(upd, jvec, best_j)
            for rot in rots:
                sh_v = jnp.take(best_val, rot)
                sh_j = jnp.take(best_j, rot)
                better = sh_v < best_val
                tie = (sh_v == best_val) & (sh_j < best_j)
                best_val = jnp.where(better, sh_v, best_val)
                best_j = jnp.where(better | tie, sh_j, best_j)
            idxs.append(best_j[0])
        cp_b.wait()
        cp_wt.wait()
        for r in range(ROWS_PER_WORKER):
            off = idxs[r] * C_Z
            for c in range(C_CHUNKS):
                out_v[pl.ds(r * C_Z + c * LANES, LANES)] = (
                    wt_v[pl.ds(off + c * LANES, LANES)]
                    + b_v[pl.ds(c * LANES, LANES)]
                )
        pltpu.async_copy(
            out_v, out_hbm.at[pl.ds(base * C_Z, ROWS_PER_WORKER * C_Z)], sem_o
        ).wait()


def kernel(residue_index, W, b):
    wt_flat = W.T.reshape(N_RES * C_Z)
    mesh = plsc.VectorSubcoreMesh(
        core_axis_name="c", subcore_axis_name="s", num_cores=1
    )
    out = pl.kernel(
        _relpos_body,
        mesh=mesh,
        out_type=jax.ShapeDtypeStruct((N_RES * C_Z,), jnp.float32),
        scratch_types=[
            pltpu.VMEM((N_PAD,), jnp.float32),
            pltpu.VMEM((C_Z,), jnp.float32),
            pltpu.VMEM((N_RES * C_Z,), jnp.float32),
            pltpu.VMEM((ROWS_PER_WORKER * C_Z,), jnp.float32),
            pltpu.SemaphoreType.DMA,
            pltpu.SemaphoreType.DMA,
            pltpu.SemaphoreType.DMA,
        ],
    )(residue_index, wt_flat, b)
    return out.reshape(N_RES, C_Z)
